# Initial kernel scaffold; baseline (speedup 1.0000x reference)
import jax
import jax.numpy as jnp
from jax.experimental import pallas as pl


def kernel(semantic_features0, semantic_features1, texture_features0, texture_features1):
    raise NotImplementedError("write your pallas kernel here")


# speedup vs baseline: 43.3688x; 43.3688x over previous
import functools

import jax
import jax.numpy as jnp
from jax.experimental import pallas as pl
from jax.experimental.pallas import tpu as pltpu

_B, _N, _M, _D = 4, 2048, 2048, 256
_BLK = 512
_NBLK = _N // _BLK


def _norm2d(x):
    nrm = jnp.sqrt(jnp.sum(x * x, axis=1, keepdims=True))
    return x / jnp.clip(nrm, 1e-12, None)


def _body(s0_ref, t0_ref, s1_ref, t1_ref,
          sim_ref, m0_ref, rmax_ref, lastcol_ref,
          cmax_ref, carg_ref, lastrow_ref,
          s1n_ref, t1n_ref):
    i = pl.program_id(1)

    @pl.when(i == 0)
    def _():
        s1n_ref[...] = s1_ref[0]
        t1n_ref[...] = t1_ref[0]

    s0n = s0_ref[0]
    t0n = t0_ref[0]
    dn = (((1,), (1,)), ((), ()))
    sim_s = jax.lax.dot_general(s0n, s1n_ref[...], dn,
                                precision=jax.lax.Precision.DEFAULT,
                                preferred_element_type=jnp.float32)
    sim_t = jax.lax.dot_general(t0n, t1n_ref[...], dn,
                                precision=jax.lax.Precision.DEFAULT,
                                preferred_element_type=jnp.float32)
    sim = sim_s * sim_t
    sim_ref[0] = sim

    iota_m = jax.lax.broadcasted_iota(jnp.int32, (_BLK, _M), 1)
    iota_n = jax.lax.broadcasted_iota(jnp.int32, (_BLK, _M), 0) + i * _BLK

    rmax = jnp.max(sim, axis=1)
    rarg = jnp.min(jnp.where(sim == rmax[:, None], iota_m, _M), axis=1)
    rmax_ref[0, 0] = rmax
    m0_ref[0, 0] = rarg
    lastcol_ref[0, 0] = sim[:, _M - 1]

    bcmax = jnp.max(sim, axis=0)
    bcarg = jnp.min(jnp.where(sim == bcmax[None, :], iota_n, _N), axis=0)

    @pl.when(i == 0)
    def _():
        cmax_ref[0, 0] = bcmax
        carg_ref[0, 0] = bcarg

    @pl.when(i > 0)
    def _():
        prev = cmax_ref[0, 0]
        prevarg = carg_ref[0, 0]
        better = bcmax > prev
        cmax_ref[0, 0] = jnp.where(better, bcmax, prev)
        carg_ref[0, 0] = jnp.where(better, bcarg, prevarg)

    @pl.when(i == _NBLK - 1)
    def _():
        lastrow_ref[0, 0] = sim[_BLK - 1, :]


@functools.partial(jax.jit, static_argnames=("interpret",))
def _matcher_core(s0, s1, t0, t1, interpret=False):
    out = pl.pallas_call(
        _body,
        grid=(_B, _NBLK),
        in_specs=[
            pl.BlockSpec((1, _BLK, _D), lambda b, i: (b, i, 0)),
            pl.BlockSpec((1, _BLK, _D), lambda b, i: (b, i, 0)),
            pl.BlockSpec((1, _M, _D), lambda b, i: (b, 0, 0)),
            pl.BlockSpec((1, _M, _D), lambda b, i: (b, 0, 0)),
        ],
        out_specs=[
            pl.BlockSpec((1, _BLK, _M), lambda b, i: (b, i, 0)),
            pl.BlockSpec((1, 1, _BLK), lambda b, i: (b * _NBLK + i, 0, 0)),
            pl.BlockSpec((1, 1, _BLK), lambda b, i: (b * _NBLK + i, 0, 0)),
            pl.BlockSpec((1, 1, _BLK), lambda b, i: (b * _NBLK + i, 0, 0)),
            pl.BlockSpec((1, 1, _M), lambda b, i: (b, 0, 0)),
            pl.BlockSpec((1, 1, _M), lambda b, i: (b, 0, 0)),
            pl.BlockSpec((1, 1, _M), lambda b, i: (b, 0, 0)),
        ],
        out_shape=[
            jax.ShapeDtypeStruct((_B, _N, _M), jnp.float32),
            jax.ShapeDtypeStruct((_B * _NBLK, 1, _BLK), jnp.int32),
            jax.ShapeDtypeStruct((_B * _NBLK, 1, _BLK), jnp.float32),
            jax.ShapeDtypeStruct((_B * _NBLK, 1, _BLK), jnp.float32),
            jax.ShapeDtypeStruct((_B, 1, _M), jnp.float32),
            jax.ShapeDtypeStruct((_B, 1, _M), jnp.int32),
            jax.ShapeDtypeStruct((_B, 1, _M), jnp.float32),
        ],
        scratch_shapes=[
            pltpu.VMEM((_M, _D), jnp.float32),
            pltpu.VMEM((_M, _D), jnp.float32),
        ],
        compiler_params=pltpu.CompilerParams(
            dimension_semantics=("arbitrary", "arbitrary"),
        ),
        interpret=interpret,
    )(s0, t0, s1, t1)
    return out


def kernel(semantic_features0, semantic_features1, texture_features0, texture_features1):
    def _nz(x):
        return x / jnp.clip(jnp.linalg.norm(x, axis=-1, keepdims=True), 1e-12, None)

    (sim, m0b, rmaxb, lastcolb, cmax, carg, lastrowb) = _matcher_core(
        _nz(semantic_features0), _nz(semantic_features1),
        _nz(texture_features0), _nz(texture_features1))
    m0raw = m0b.reshape(_B, _N)
    rowmax = rmaxb.reshape(_B, _N)
    lastcol = lastcolb.reshape(_B, _N)
    m1raw = carg.reshape(_B, _M)
    colmax = cmax.reshape(_B, _M)
    lastrow = lastrowb.reshape(_B, _M)

    loop0 = jnp.take_along_axis(m1raw, m0raw, axis=-1)
    loop1 = jnp.take_along_axis(m0raw, m1raw, axis=-1)
    inds0 = jnp.arange(_N)[None, :]
    inds1 = jnp.arange(_M)[None, :]
    mut0 = inds0 == loop0
    mut1 = inds1 == loop1
    matches0 = jnp.where(mut0, m0raw, -1)
    matches1 = jnp.where(mut1, m1raw, -1)
    mscores0 = jnp.where(mut0, rowmax, lastcol)
    mscores1 = jnp.where(mut1, colmax, lastrow)
    return matches0, matches1, mscores0, mscores1, sim


# speedup vs baseline: 53.8622x; 1.2420x over previous
import functools

import jax
import jax.numpy as jnp
from jax.experimental import pallas as pl
from jax.experimental.pallas import tpu as pltpu

_B, _N, _M, _D = 4, 2048, 2048, 256
_BLK = 512
_NBLK = _N // _BLK


def _norm2d(x):
    nrm = jnp.sqrt(jnp.sum(x * x, axis=1, keepdims=True))
    return x / jnp.clip(nrm, 1e-12, None)


def _body(s0_ref, t0_ref, s1_ref, t1_ref,
          sim_ref, m0_ref, rmax_ref, lastcol_ref,
          cmax_ref, carg_ref, lastrow_ref,
          s1n_ref, t1n_ref):
    i = pl.program_id(1)

    @pl.when(i == 0)
    def _():
        s1n_ref[...] = _norm2d(s1_ref[0])
        t1n_ref[...] = _norm2d(t1_ref[0])

    s0n = _norm2d(s0_ref[0])
    t0n = _norm2d(t0_ref[0])
    dn = (((1,), (1,)), ((), ()))
    sim_s = jax.lax.dot_general(s0n, s1n_ref[...], dn,
                                precision=jax.lax.Precision.DEFAULT,
                                preferred_element_type=jnp.float32)
    sim_t = jax.lax.dot_general(t0n, t1n_ref[...], dn,
                                precision=jax.lax.Precision.DEFAULT,
                                preferred_element_type=jnp.float32)
    sim = sim_s * sim_t
    sim_ref[0] = sim

    iota_m = jax.lax.broadcasted_iota(jnp.int32, (_BLK, _M), 1)
    iota_n = jax.lax.broadcasted_iota(jnp.int32, (_BLK, _M), 0) + i * _BLK

    rmax = jnp.max(sim, axis=1)
    rarg = jnp.min(jnp.where(sim == rmax[:, None], iota_m, _M), axis=1)
    rmax_ref[0, 0] = rmax
    m0_ref[0, 0] = rarg
    lastcol_ref[0, 0] = sim[:, _M - 1]

    bcmax = jnp.max(sim, axis=0)
    bcarg = jnp.min(jnp.where(sim == bcmax[None, :], iota_n, _N), axis=0)

    @pl.when(i == 0)
    def _():
        cmax_ref[0, 0] = bcmax
        carg_ref[0, 0] = bcarg

    @pl.when(i > 0)
    def _():
        prev = cmax_ref[0, 0]
        prevarg = carg_ref[0, 0]
        better = bcmax > prev
        cmax_ref[0, 0] = jnp.where(better, bcmax, prev)
        carg_ref[0, 0] = jnp.where(better, bcarg, prevarg)

    @pl.when(i == _NBLK - 1)
    def _():
        lastrow_ref[0, 0] = sim[_BLK - 1, :]


@functools.partial(jax.jit, static_argnames=("interpret",))
def _matcher_core(s0, s1, t0, t1, interpret=False):
    out = pl.pallas_call(
        _body,
        grid=(_B, _NBLK),
        in_specs=[
            pl.BlockSpec((1, _BLK, _D), lambda b, i: (b, i, 0)),
            pl.BlockSpec((1, _BLK, _D), lambda b, i: (b, i, 0)),
            pl.BlockSpec((1, _M, _D), lambda b, i: (b, 0, 0)),
            pl.BlockSpec((1, _M, _D), lambda b, i: (b, 0, 0)),
        ],
        out_specs=[
            pl.BlockSpec((1, _BLK, _M), lambda b, i: (b, i, 0)),
            pl.BlockSpec((1, 1, _BLK), lambda b, i: (b * _NBLK + i, 0, 0)),
            pl.BlockSpec((1, 1, _BLK), lambda b, i: (b * _NBLK + i, 0, 0)),
            pl.BlockSpec((1, 1, _BLK), lambda b, i: (b * _NBLK + i, 0, 0)),
            pl.BlockSpec((1, 1, _M), lambda b, i: (b, 0, 0)),
            pl.BlockSpec((1, 1, _M), lambda b, i: (b, 0, 0)),
            pl.BlockSpec((1, 1, _M), lambda b, i: (b, 0, 0)),
        ],
        out_shape=[
            jax.ShapeDtypeStruct((_B, _N, _M), jnp.float32),
            jax.ShapeDtypeStruct((_B * _NBLK, 1, _BLK), jnp.int32),
            jax.ShapeDtypeStruct((_B * _NBLK, 1, _BLK), jnp.float32),
            jax.ShapeDtypeStruct((_B * _NBLK, 1, _BLK), jnp.float32),
            jax.ShapeDtypeStruct((_B, 1, _M), jnp.float32),
            jax.ShapeDtypeStruct((_B, 1, _M), jnp.int32),
            jax.ShapeDtypeStruct((_B, 1, _M), jnp.float32),
        ],
        scratch_shapes=[
            pltpu.VMEM((_M, _D), jnp.float32),
            pltpu.VMEM((_M, _D), jnp.float32),
        ],
        compiler_params=pltpu.CompilerParams(
            dimension_semantics=("arbitrary", "arbitrary"),
        ),
        interpret=interpret,
    )(s0, t0, s1, t1)
    return out


def kernel(semantic_features0, semantic_features1, texture_features0, texture_features1):
    (sim, m0b, rmaxb, lastcolb, cmax, carg, lastrowb) = _matcher_core(
        semantic_features0, semantic_features1,
        texture_features0, texture_features1)
    m0raw = m0b.reshape(_B, _N)
    rowmax = rmaxb.reshape(_B, _N)
    lastcol = lastcolb.reshape(_B, _N)
    m1raw = carg.reshape(_B, _M)
    colmax = cmax.reshape(_B, _M)
    lastrow = lastrowb.reshape(_B, _M)

    loop0 = jnp.take_along_axis(m1raw, m0raw, axis=-1)
    loop1 = jnp.take_along_axis(m0raw, m1raw, axis=-1)
    inds0 = jnp.arange(_N)[None, :]
    inds1 = jnp.arange(_M)[None, :]
    mut0 = inds0 == loop0
    mut1 = inds1 == loop1
    matches0 = jnp.where(mut0, m0raw, -1)
    matches1 = jnp.where(mut1, m1raw, -1)
    mscores0 = jnp.where(mut0, rowmax, lastcol)
    mscores1 = jnp.where(mut1, colmax, lastrow)
    return matches0, matches1, mscores0, mscores1, sim
